# baseline (speedup 1.0000x reference)
import functools

import jax
import jax.numpy as jnp
from jax import lax
from jax.experimental import pallas as pl
from jax.experimental.pallas import tpu as pltpu
from jax.experimental.pallas import tpu_sc as plsc

N_FIELDS = 26
VOCAB = 1000
EMB_DIM = 1000
BATCH = 1024
ROWS = BATCH * N_FIELDS
NC, NS, L = 2, 16, 16
NW = NC * NS
ROWS_PER_W = ROWS // NW
CHUNK = 32
NCHUNK = ROWS_PER_W // CHUNK
NSEG = 8
TAIL_COL = (NSEG - 1) * 128
OUT_MINOR = NSEG * 128
NGROUP = ROWS_PER_W // L


def _make_gather():
    mesh = plsc.VectorSubcoreMesh(core_axis_name="c", subcore_axis_name="s")

    @functools.partial(
        pl.kernel,
        mesh=mesh,
        out_type=jax.ShapeDtypeStruct((ROWS, OUT_MINOR), jnp.float32),
        scratch_types=[
            pltpu.VMEM((ROWS_PER_W,), jnp.int32),
            pltpu.VMEM((ROWS_PER_W,), jnp.int32),
            pltpu.VMEM((NSEG, CHUNK, 128), jnp.float32),
            pltpu.VMEM((NSEG, CHUNK, 128), jnp.float32),
            pltpu.SemaphoreType.DMA,
            pltpu.SemaphoreType.DMA,
            pltpu.SemaphoreType.DMA,
            pltpu.SemaphoreType.DMA,
        ],
        compiler_params=pltpu.CompilerParams(use_tc_tiling_on_sc=True,
                                             needs_layout_passes=False),
    )
    def gather_kernel(table, aux, x_hbm, out, x_v, idx_v,
                      buf0, buf1, gsem0, gsem1, ssem0, ssem1):
        wid = lax.axis_index("s") * NC + lax.axis_index("c")
        base_row = wid * ROWS_PER_W
        pltpu.sync_copy(x_hbm.at[pl.ds(base_row, ROWS_PER_W)], x_v)

        lane = lax.broadcasted_iota(jnp.int32, (L,), 0)
        zero16i = jnp.zeros((L,), jnp.int32)
        zeros16 = jnp.zeros((L,), jnp.float32)
        cvocab = jnp.full((L,), VOCAB, jnp.int32)
        c16 = jnp.full((L,), L, jnp.int32)
        c26 = jnp.full((L,), N_FIELDS, jnp.int32)

        def idx_body(j, fld):
            v = x_v[pl.ds(j * L, L)]
            idx_v[pl.ds(j * L, L)] = v + fld * cvocab
            t = fld + c16
            return jnp.where(t >= c26, t - c26, t)

        lax.fori_loop(0, NGROUP, idx_body, lane)

        bufs = (buf0, buf1)
        gsems = (gsem0, gsem1)
        ssems = (ssem0, ssem1)

        def issue_gathers(c, b):
            off = pl.multiple_of(c * CHUNK, CHUNK)
            idx_sl = idx_v.at[pl.ds(off, CHUNK)]
            for seg in range(NSEG - 1):
                pltpu.async_copy(table.at[idx_sl, pl.ds(seg * 128, 128)],
                                 bufs[b].at[seg], gsems[b])
            pltpu.async_copy(aux.at[idx_sl], bufs[b].at[NSEG - 1], gsems[b])

        def wait_gathers(c, b):
            off = pl.multiple_of(c * CHUNK, CHUNK)
            idx_sl = idx_v.at[pl.ds(off, CHUNK)]
            for seg in range(NSEG - 1):
                pltpu.make_async_copy(
                    table.at[idx_sl, pl.ds(seg * 128, 128)],
                    bufs[b].at[seg], gsems[b]).wait()
            pltpu.make_async_copy(aux.at[idx_sl], bufs[b].at[NSEG - 1],
                                  gsems[b]).wait()

        def zero_pad_rows(c, b):
            for g in range(CHUNK // L):
                xv = x_v[pl.ds(c * CHUNK + g * L, L)]
                min_x = jnp.min(xv)

                @pl.when(min_x == 0)
                def _zero(g=g, xv=xv, b=b):
                    pad = xv == zero16i
                    rows = g * L + lane

                    def body(col, carry):
                        cols = jnp.full((L,), col, jnp.int32)
                        for seg in range(NSEG):
                            plsc.store_scatter(bufs[b].at[seg], [rows, cols],
                                               zeros16, mask=pad)
                        return carry

                    lax.fori_loop(0, 128, body, 0)

        def scatters(c, b, issue):
            r0 = pl.multiple_of(base_row + c * CHUNK, CHUNK)
            for seg in range(NSEG):
                src = bufs[b].at[seg]
                dst = out.at[pl.ds(r0, CHUNK), pl.ds(seg * 128, 128)]
                if issue:
                    pltpu.async_copy(src, dst, ssems[b])
                else:
                    pltpu.make_async_copy(src, dst, ssems[b]).wait()

        issue_gathers(0, 0)
        issue_gathers(1, 1)

        def chunk_body(k, carry):
            for sub in range(2):
                c = 2 * k + sub
                wait_gathers(c, sub)
                zero_pad_rows(c, sub)
                scatters(c, sub, True)
                scatters(c, sub, False)

                @pl.when(c + 2 < NCHUNK)
                def _prefetch(c=c, sub=sub):
                    issue_gathers(c + 2, sub)

            return carry

        lax.fori_loop(0, NCHUNK // 2, chunk_body, 0)

    return gather_kernel


_gather = _make_gather()


def kernel(x, tables):
    table_flat = tables.reshape(N_FIELDS * VOCAB, EMB_DIM)
    aux = jnp.concatenate(
        [tables[:, :, TAIL_COL:],
         jnp.zeros((N_FIELDS, VOCAB, OUT_MINOR - EMB_DIM), jnp.float32)],
        axis=-1)
    aux_flat = aux.reshape(N_FIELDS * VOCAB, 128)
    x_flat = x.reshape(ROWS)
    out = _gather(table_flat, aux_flat, x_flat)
    return out[:, :EMB_DIM].reshape(BATCH, N_FIELDS * EMB_DIM)


# speedup vs baseline: 1.0071x; 1.0071x over previous
import functools

import jax
import jax.numpy as jnp
from jax import lax
from jax.experimental import pallas as pl
from jax.experimental.pallas import tpu as pltpu
from jax.experimental.pallas import tpu_sc as plsc

N_FIELDS = 26
VOCAB = 1000
EMB_DIM = 1000
BATCH = 1024
ROWS = BATCH * N_FIELDS
NC, NS, L = 2, 16, 16
NW = NC * NS
ROWS_PER_W = ROWS // NW
CHUNK = 32
NCHUNK = ROWS_PER_W // CHUNK
NSEG = 8
TAIL_COL = (NSEG - 1) * 128
OUT_MINOR = NSEG * 128
NGROUP = ROWS_PER_W // L


def _make_gather():
    mesh = plsc.VectorSubcoreMesh(core_axis_name="c", subcore_axis_name="s")

    @functools.partial(
        pl.kernel,
        mesh=mesh,
        out_type=jax.ShapeDtypeStruct((ROWS, OUT_MINOR), jnp.float32),
        scratch_types=[
            pltpu.VMEM((ROWS_PER_W,), jnp.int32),
            pltpu.VMEM((ROWS_PER_W,), jnp.int32),
            pltpu.VMEM((CHUNK, TAIL_COL), jnp.float32),
            pltpu.VMEM((CHUNK, TAIL_COL), jnp.float32),
            pltpu.VMEM((CHUNK, 128), jnp.float32),
            pltpu.VMEM((CHUNK, 128), jnp.float32),
            pltpu.SemaphoreType.DMA,
            pltpu.SemaphoreType.DMA,
            pltpu.SemaphoreType.DMA,
            pltpu.SemaphoreType.DMA,
        ],
        compiler_params=pltpu.CompilerParams(use_tc_tiling_on_sc=True,
                                             needs_layout_passes=False),
    )
    def gather_kernel(table, aux, x_hbm, out, x_v, idx_v,
                      buf0, buf1, tbuf0, tbuf1, gsem0, gsem1, ssem0, ssem1):
        wid = lax.axis_index("s") * NC + lax.axis_index("c")
        base_row = wid * ROWS_PER_W
        pltpu.sync_copy(x_hbm.at[pl.ds(base_row, ROWS_PER_W)], x_v)

        lane = lax.broadcasted_iota(jnp.int32, (L,), 0)
        zero16i = jnp.zeros((L,), jnp.int32)
        zeros16 = jnp.zeros((L,), jnp.float32)
        cvocab = jnp.full((L,), VOCAB, jnp.int32)
        c16 = jnp.full((L,), L, jnp.int32)
        c26 = jnp.full((L,), N_FIELDS, jnp.int32)

        def idx_body(j, fld):
            v = x_v[pl.ds(j * L, L)]
            idx_v[pl.ds(j * L, L)] = v + fld * cvocab
            t = fld + c16
            return jnp.where(t >= c26, t - c26, t)

        lax.fori_loop(0, NGROUP, idx_body, lane)

        bufs = (buf0, buf1)
        tbufs = (tbuf0, tbuf1)
        gsems = (gsem0, gsem1)
        ssems = (ssem0, ssem1)

        def issue_gathers(c, b):
            off = pl.multiple_of(c * CHUNK, CHUNK)
            idx_sl = idx_v.at[pl.ds(off, CHUNK)]
            pltpu.async_copy(table.at[idx_sl, pl.ds(0, TAIL_COL)],
                             bufs[b], gsems[b])
            pltpu.async_copy(aux.at[idx_sl], tbufs[b], gsems[b])

        def wait_gathers(c, b):
            off = pl.multiple_of(c * CHUNK, CHUNK)
            idx_sl = idx_v.at[pl.ds(off, CHUNK)]
            pltpu.make_async_copy(table.at[idx_sl, pl.ds(0, TAIL_COL)],
                                  bufs[b], gsems[b]).wait()
            pltpu.make_async_copy(aux.at[idx_sl], tbufs[b],
                                  gsems[b]).wait()

        def zero_pad_rows(c, b):
            for g in range(CHUNK // L):
                xv = x_v[pl.ds(c * CHUNK + g * L, L)]
                min_x = jnp.min(xv)

                @pl.when(min_x == 0)
                def _zero(g=g, xv=xv, b=b):
                    pad = xv == zero16i
                    rows = g * L + lane

                    def body(col, carry):
                        cols = jnp.full((L,), col, jnp.int32)
                        plsc.store_scatter(tbufs[b], [rows, cols],
                                           zeros16, mask=pad)
                        for rep in range(NSEG - 1):
                            cols7 = cols + jnp.full((L,), rep * 128,
                                                    jnp.int32)
                            plsc.store_scatter(bufs[b], [rows, cols7],
                                               zeros16, mask=pad)
                        return carry

                    lax.fori_loop(0, 128, body, 0)

        def scatters(c, b, issue):
            r0 = pl.multiple_of(base_row + c * CHUNK, CHUNK)
            pairs = [(bufs[b], out.at[pl.ds(r0, CHUNK), pl.ds(0, TAIL_COL)]),
                     (tbufs[b],
                      out.at[pl.ds(r0, CHUNK), pl.ds(TAIL_COL, 128)])]
            for src, dst in pairs:
                if issue:
                    pltpu.async_copy(src, dst, ssems[b])
                else:
                    pltpu.make_async_copy(src, dst, ssems[b]).wait()

        issue_gathers(0, 0)
        issue_gathers(1, 1)

        def chunk_body(k, carry):
            for sub in range(2):
                c = 2 * k + sub
                wait_gathers(c, sub)
                zero_pad_rows(c, sub)
                scatters(c, sub, True)
                scatters(c, sub, False)

                @pl.when(c + 2 < NCHUNK)
                def _prefetch(c=c, sub=sub):
                    issue_gathers(c + 2, sub)

            return carry

        lax.fori_loop(0, NCHUNK // 2, chunk_body, 0)

    return gather_kernel


_gather = _make_gather()


def kernel(x, tables):
    table_flat = tables.reshape(N_FIELDS * VOCAB, EMB_DIM)
    aux = jnp.concatenate(
        [tables[:, :, TAIL_COL:],
         jnp.zeros((N_FIELDS, VOCAB, OUT_MINOR - EMB_DIM), jnp.float32)],
        axis=-1)
    aux_flat = aux.reshape(N_FIELDS * VOCAB, 128)
    x_flat = x.reshape(ROWS)
    out = _gather(table_flat, aux_flat, x_flat)
    return out[:, :EMB_DIM].reshape(BATCH, N_FIELDS * EMB_DIM)
